# baseline (speedup 1.0000x reference)
import functools

import jax
import jax.numpy as jnp
from jax import lax
from jax.experimental import pallas as pl
from jax.experimental.pallas import tpu as pltpu
from jax.experimental.pallas import tpu_sc as plsc

E = 320000
N = 10000
D = 128

_NC, _NS = 2, 16
_NW = _NC * _NS
_CPT = 80
_EP = _NW * _CPT * 128
_CHUNKS = _EP // 128


_BLK = 2560


_GRID_E = E // _BLK
_GRID = _GRID_E + (_EP - E) // _BLK


def _mlp_body(b2_ref, f_ref, v_ref, w1_ref, b1_ref, w2_ref, out_ref):
    i = pl.program_id(0)
    f = f_ref[...].astype(jnp.bfloat16)
    w1 = w1_ref[...].astype(jnp.bfloat16)
    h = jnp.dot(f, w1, preferred_element_type=jnp.float32) + b1_ref[...]
    h = (h * jax.nn.sigmoid(h)).astype(jnp.bfloat16)
    o8 = jnp.dot(h, w2_ref[...].astype(jnp.bfloat16),
                 preferred_element_type=jnp.float32) + b2_ref[0]
    out = o8 * v_ref[...]
    out_ref[...] = jnp.where(i < _GRID_E, out, 0.0)


def _edge_mlp(forces, V8, W1, b1r, w2r, b2):
    clamp = lambda i: (jnp.minimum(i, _GRID_E - 1), 0)
    return pl.pallas_call(
        _mlp_body,
        grid=_GRID,
        in_specs=[
            pl.BlockSpec(memory_space=pltpu.SMEM),
            pl.BlockSpec((_BLK, D), clamp),
            pl.BlockSpec((_BLK, 8), clamp),
            pl.BlockSpec((D, D), lambda i: (0, 0)),
            pl.BlockSpec((1, D), lambda i: (0, 0)),
            pl.BlockSpec((D, 8), lambda i: (0, 0)),
        ],
        out_specs=pl.BlockSpec((_BLK, 8), lambda i: (i, 0)),
        out_shape=jax.ShapeDtypeStruct((_EP, 8), jnp.float32),
        compiler_params=pltpu.CompilerParams(
            dimension_semantics=("arbitrary",)),
    )(b2, forces, V8, W1, b1r, w2r)


_sc_mesh = plsc.VectorSubcoreMesh(
    core_axis_name="c", subcore_axis_name="s",
    num_cores=_NC, num_subcores=_NS)


@functools.partial(
    pl.kernel,
    mesh=_sc_mesh,
    out_type=jax.ShapeDtypeStruct((_NC, N, 8), jnp.float32),
    scratch_types=[
        pltpu.VMEM((_CPT, 128), jnp.int32),
        pltpu.VMEM((_CPT * 128, 8), jnp.float32),
        pltpu.VMEM_SHARED((N, 8), jnp.float32),
        pltpu.SemaphoreType.DMA,
    ],
    compiler_params=pltpu.CompilerParams(use_tc_tiling_on_sc=False),
)
def _sc_scatter(vals_hbm, idx_hbm, zeros_hbm, out_hbm, idx_v, vals_v, acc, sem):
    cid = lax.axis_index("c")
    sid = lax.axis_index("s")
    wid = cid * _NS + sid

    @pl.when(sid == 0)
    def _():
        pltpu.sync_copy(zeros_hbm, acc)

    pltpu.sync_copy(idx_hbm.at[pl.ds(wid * _CPT, _CPT)], idx_v)
    pltpu.sync_copy(vals_hbm.at[pl.ds(wid * (_CPT * 128), _CPT * 128)], vals_v)

    plsc.subcore_barrier()

    K = 8

    def body(g, carry):
        descs = []
        for k in range(K):
            j = g * K + k
            descs.append(pltpu.async_copy(
                vals_v.at[pl.ds(j * 128, 128)],
                acc.at[idx_v.at[j]], sem, add=True))
        for d in descs:
            d.wait()
        return carry

    lax.fori_loop(0, _CPT // K, body, 0)

    plsc.subcore_barrier()

    @pl.when(sid == 0)
    def _():
        pltpu.sync_copy(acc, out_hbm.at[cid])


def kernel(forces, V_st, idx_t, atomic_numbers, W1, b1, W2, b2):
    V8 = jnp.concatenate(
        [V_st, jnp.zeros((E, 5), V_st.dtype)], axis=1)
    vals = _edge_mlp(forces, V8, W1,
                     b1.reshape(1, D), jnp.tile(W2, (1, 8)),
                     b2.astype(jnp.float32))
    idx2d = jnp.concatenate(
        [idx_t.astype(jnp.int32), jnp.zeros((_EP - E,), jnp.int32)]
    ).reshape(_CHUNKS, 128)
    zeros = jnp.zeros((N, 8), jnp.float32)
    parts = _sc_scatter(vals, idx2d, zeros)
    return (parts[0] + parts[1])[:, :3]


# speedup vs baseline: 1.2337x; 1.2337x over previous
import functools

import jax
import jax.numpy as jnp
from jax import lax
from jax.experimental import pallas as pl
from jax.experimental.pallas import tpu as pltpu
from jax.experimental.pallas import tpu_sc as plsc

E = 320000
N = 10000
D = 128

_NC, _NS = 2, 16
_NW = _NC * _NS
_CPT = 80
_EP = _NW * _CPT * 128
_CHUNKS = _EP // 128


_BLK = 2560


_GRID_E = E // _BLK
_GRID = _GRID_E + (_EP - E) // _BLK


def _mlp_body(b2_ref, f_ref, v_ref, w1_ref, b1_ref, w2_ref, out_ref):
    i = pl.program_id(0)
    f = f_ref[...].astype(jnp.bfloat16)
    w1 = w1_ref[...].astype(jnp.bfloat16)
    h = jnp.dot(f, w1, preferred_element_type=jnp.float32) + b1_ref[...]
    h = (h * jax.nn.sigmoid(h)).astype(jnp.bfloat16)
    o8 = jnp.dot(h, w2_ref[...].astype(jnp.bfloat16),
                 preferred_element_type=jnp.float32) + b2_ref[0]
    v8 = jnp.concatenate(
        [v_ref[...], jnp.zeros((_BLK, 5), jnp.float32)], axis=1)
    out = o8 * v8
    out_ref[...] = jnp.where(i < _GRID_E, out, 0.0)


def _edge_mlp(forces, V_st, W1, b1r, w2r, b2):
    clamp = lambda i: (jnp.minimum(i, _GRID_E - 1), 0)
    return pl.pallas_call(
        _mlp_body,
        grid=_GRID,
        in_specs=[
            pl.BlockSpec(memory_space=pltpu.SMEM),
            pl.BlockSpec((_BLK, D), clamp),
            pl.BlockSpec((_BLK, 3), clamp),
            pl.BlockSpec((D, D), lambda i: (0, 0)),
            pl.BlockSpec((1, D), lambda i: (0, 0)),
            pl.BlockSpec((D, 8), lambda i: (0, 0)),
        ],
        out_specs=pl.BlockSpec((_BLK, 8), lambda i: (i, 0)),
        out_shape=jax.ShapeDtypeStruct((_EP, 8), jnp.float32),
        compiler_params=pltpu.CompilerParams(
            dimension_semantics=("arbitrary",)),
    )(b2, forces, V_st, W1, b1r, w2r)


_sc_mesh = plsc.VectorSubcoreMesh(
    core_axis_name="c", subcore_axis_name="s",
    num_cores=_NC, num_subcores=_NS)


@functools.partial(
    pl.kernel,
    mesh=_sc_mesh,
    out_type=jax.ShapeDtypeStruct((_NC, N, 8), jnp.float32),
    scratch_types=[
        pltpu.VMEM((_CPT, 128), jnp.int32),
        pltpu.VMEM((_CPT * 128, 8), jnp.float32),
        pltpu.VMEM_SHARED((N, 8), jnp.float32),
        pltpu.SemaphoreType.DMA,
    ],
    compiler_params=pltpu.CompilerParams(use_tc_tiling_on_sc=False),
)
def _sc_scatter(vals_hbm, idx_hbm, zeros_hbm, out_hbm, idx_v, vals_v, acc, sem):
    cid = lax.axis_index("c")
    sid = lax.axis_index("s")
    wid = cid * _NS + sid

    @pl.when(sid == 0)
    def _():
        pltpu.sync_copy(zeros_hbm, acc)

    pltpu.sync_copy(idx_hbm.at[pl.ds(wid * _CPT, _CPT)], idx_v)
    pltpu.sync_copy(vals_hbm.at[pl.ds(wid * (_CPT * 128), _CPT * 128)], vals_v)

    plsc.subcore_barrier()

    K = 8

    def body(g, carry):
        descs = []
        for k in range(K):
            j = g * K + k
            descs.append(pltpu.async_copy(
                vals_v.at[pl.ds(j * 128, 128)],
                acc.at[idx_v.at[j]], sem, add=True))
        for d in descs:
            d.wait()
        return carry

    lax.fori_loop(0, _CPT // K, body, 0)

    plsc.subcore_barrier()

    @pl.when(sid == 0)
    def _():
        pltpu.sync_copy(acc, out_hbm.at[cid])


def kernel(forces, V_st, idx_t, atomic_numbers, W1, b1, W2, b2):
    vals = _edge_mlp(forces, V_st, W1,
                     b1.reshape(1, D), jnp.tile(W2, (1, 8)),
                     b2.astype(jnp.float32))
    idx2d = jnp.concatenate(
        [idx_t.astype(jnp.int32), jnp.zeros((_EP - E,), jnp.int32)]
    ).reshape(_CHUNKS, 128)
    zeros = jnp.zeros((N, 8), jnp.float32)
    parts = _sc_scatter(vals, idx2d, zeros)
    return (parts[0] + parts[1])[:, :3]
